# baseline (speedup 1.0000x reference)
import functools

import jax
import jax.numpy as jnp
from jax import lax
from jax.experimental import pallas as pl
from jax.experimental.pallas import tpu as pltpu
from jax.experimental.pallas import tpu_sc as plsc

NC = 2
NS = 16
NW = NC * NS
L = 16

R = 12800
B = 4096
RPW = R // NW
RC = 8
NSTEP = RPW // RC
COLV = B // L


def _compute(buf, pbv, g):
    for r in range(RC):
        splat = pbv[pl.ds((g * RC + r) * L, L)]

        @pl.loop(0, COLV, unroll=8)
        def _(i, splat=splat, r=r):
            plsc.addupdate(buf.at[r, pl.ds(i * L, L)], splat)


def _sc_body(x_hbm, pb_hbm, out_hbm, buf0, buf1, pb_v, si0, si1, so0, so1):
    c = lax.axis_index("c")
    s = lax.axis_index("s")
    w = s * NC + c
    row0 = w * RPW
    pltpu.sync_copy(pb_hbm.at[pl.ds(row0 * L, RPW * L)], pb_v)

    bufs = (buf0, buf1)
    sin = (si0, si1)
    sout = (so0, so1)

    def in_cp(g, b):
        return pltpu.make_async_copy(
            x_hbm.at[pl.ds(row0 + g * RC, RC)], bufs[b], sin[b])

    def out_cp(g, b):
        return pltpu.make_async_copy(
            bufs[b], out_hbm.at[pl.ds(row0 + g * RC, RC)], sout[b])

    in_cp(0, 0).start()

    @pl.loop(0, NSTEP, step=2)
    def _(g0):
        for b in range(2):
            g = g0 + b

            @pl.when(g >= 1)
            def _(g=g, b=b):
                out_cp(g - 1, 1 - b).wait()

            @pl.when(g + 1 < NSTEP)
            def _(g=g, b=b):
                in_cp(g + 1, 1 - b).start()

            in_cp(g, b).wait()
            _compute(bufs[b], pb_v, g)
            out_cp(g, b).start()

    out_cp(NSTEP - 1, 1).wait()


_sc_call = functools.partial(
    pl.kernel,
    out_type=jax.ShapeDtypeStruct((R, B), jnp.float32),
    mesh=plsc.VectorSubcoreMesh(core_axis_name="c", subcore_axis_name="s",
                                num_cores=NC, num_subcores=NS),
    scratch_types=[
        pltpu.VMEM((RC, B), jnp.float32),
        pltpu.VMEM((RC, B), jnp.float32),
        pltpu.VMEM((RPW * L,), jnp.float32),
        pltpu.SemaphoreType.DMA,
        pltpu.SemaphoreType.DMA,
        pltpu.SemaphoreType.DMA,
        pltpu.SemaphoreType.DMA,
    ],
)(_sc_body)


def kernel(x, pos_table):
    Bx, n, d = x.shape
    x2 = jnp.transpose(x, (1, 2, 0)).reshape(R, B)
    posf = pos_table[:n].reshape(R)
    pb16 = jnp.repeat(posf, L)
    out2 = _sc_call(x2, pb16)
    return jnp.transpose(out2.reshape(n, d, Bx), (2, 0, 1))


# speedup vs baseline: 1.0015x; 1.0015x over previous
import functools

import jax
import jax.numpy as jnp
from jax import lax
from jax.experimental import pallas as pl
from jax.experimental.pallas import tpu as pltpu
from jax.experimental.pallas import tpu_sc as plsc

NC = 2
NS = 16
NW = NC * NS
L = 16

R = 12800
B = 4096
RPW = R // NW
RC = 8
NSTEP = RPW // RC
COLV = B // L


def _compute(buf, pbv, g):
    for r in range(RC):
        splat = pbv[pl.ds((g * RC + r) * L, L)]

        @pl.loop(0, COLV, unroll=8)
        def _(i, splat=splat, r=r):
            plsc.addupdate(buf.at[r, pl.ds(i * L, L)], splat)


def _sc_body(x_hbm, pb_hbm, out_hbm, buf0, buf1, pb_v, si0, si1, so0, so1):
    c = lax.axis_index("c")
    s = lax.axis_index("s")
    w = s * NC + c
    row0 = w * RPW
    pltpu.sync_copy(pb_hbm.at[pl.ds(row0 * L, RPW * L)], pb_v)

    bufs = (buf0, buf1)
    sin = (si0, si1)
    sout = (so0, so1)

    def in_cp(g, b):
        return pltpu.make_async_copy(
            x_hbm.at[pl.ds(row0 + g * RC, RC)], bufs[b], sin[b])

    def out_cp(g, b):
        return pltpu.make_async_copy(
            bufs[b], out_hbm.at[pl.ds(row0 + g * RC, RC)], sout[b])

    in_cp(0, 0).start()

    @pl.loop(0, NSTEP, step=2)
    def _(g0):
        for b in range(2):
            g = g0 + b

            @pl.when(g >= 1)
            def _(g=g, b=b):
                out_cp(g - 1, 1 - b).wait()

            @pl.when(g + 1 < NSTEP)
            def _(g=g, b=b):
                in_cp(g + 1, 1 - b).start()

            in_cp(g, b).wait()
            _compute(bufs[b], pb_v, g)
            out_cp(g, b).start()

    out_cp(NSTEP - 1, 1).wait()


_sc_call_cache = []


def _get_sc_call():
    if not _sc_call_cache:
        _sc_call_cache.append(functools.partial(
            pl.kernel,
            out_type=jax.ShapeDtypeStruct((R, B), jnp.float32),
            mesh=plsc.VectorSubcoreMesh(
                core_axis_name="c", subcore_axis_name="s",
                num_cores=NC, num_subcores=NS),
            scratch_types=[
                pltpu.VMEM((RC, B), jnp.float32),
                pltpu.VMEM((RC, B), jnp.float32),
                pltpu.VMEM((RPW * L,), jnp.float32),
                pltpu.SemaphoreType.DMA,
                pltpu.SemaphoreType.DMA,
                pltpu.SemaphoreType.DMA,
                pltpu.SemaphoreType.DMA,
            ],
        )(_sc_body))
    return _sc_call_cache[0]


def kernel(x, pos_table):
    Bx, n, d = x.shape
    x2 = jnp.transpose(x, (1, 2, 0)).reshape(R, B)
    posf = pos_table[:n].reshape(R)
    pb16 = jnp.repeat(posf, L)
    out2 = _get_sc_call()(x2, pb16)
    return jnp.transpose(out2.reshape(n, d, Bx), (2, 0, 1))


# speedup vs baseline: 1.0115x; 1.0101x over previous
import functools

import jax
import jax.numpy as jnp
from jax import lax
from jax.experimental import pallas as pl
from jax.experimental.pallas import tpu as pltpu
from jax.experimental.pallas import tpu_sc as plsc

NC = 2
NS = 16
NW = NC * NS
L = 16

R = 12800
B = 4096
RPW = R // NW
RC = 8
NSTEP = RPW // RC
COLV = B // L


def _compute(buf, pbv, g):
    for r in range(RC):
        splat = pbv[pl.ds((g * RC + r) * L, L)]

        @pl.loop(0, COLV, unroll=8)
        def _(i, splat=splat, r=r):
            plsc.addupdate(buf.at[r, pl.ds(i * L, L)], splat)


def _sc_body(x_hbm, pb_hbm, out_hbm, buf0, buf1, pb_v, si0, si1, so0, so1):
    c = lax.axis_index("c")
    s = lax.axis_index("s")
    w = s * NC + c
    row0 = w * RPW

    bufs = (buf0, buf1)
    sin = (si0, si1)
    sout = (so0, so1)

    def in_cp(g, b):
        return pltpu.make_async_copy(
            x_hbm.at[pl.ds(row0 + g * RC, RC)], bufs[b], sin[b])

    def out_cp(g, b):
        return pltpu.make_async_copy(
            bufs[b], out_hbm.at[pl.ds(row0 + g * RC, RC)], sout[b])

    in_cp(0, 0).start()
    pltpu.sync_copy(pb_hbm.at[pl.ds(row0 * L, RPW * L)], pb_v)

    @pl.loop(0, NSTEP, step=2)
    def _(g0):
        for b in range(2):
            g = g0 + b

            @pl.when(g >= 1)
            def _(g=g, b=b):
                out_cp(g - 1, 1 - b).wait()

            @pl.when(g + 1 < NSTEP)
            def _(g=g, b=b):
                in_cp(g + 1, 1 - b).start()

            in_cp(g, b).wait()
            _compute(bufs[b], pb_v, g)
            out_cp(g, b).start()

    out_cp(NSTEP - 1, 1).wait()


_sc_call_cache = []


def _get_sc_call():
    if not _sc_call_cache:
        _sc_call_cache.append(functools.partial(
            pl.kernel,
            out_type=jax.ShapeDtypeStruct((R, B), jnp.float32),
            mesh=plsc.VectorSubcoreMesh(
                core_axis_name="c", subcore_axis_name="s",
                num_cores=NC, num_subcores=NS),
            scratch_types=[
                pltpu.VMEM((RC, B), jnp.float32),
                pltpu.VMEM((RC, B), jnp.float32),
                pltpu.VMEM((RPW * L,), jnp.float32),
                pltpu.SemaphoreType.DMA,
                pltpu.SemaphoreType.DMA,
                pltpu.SemaphoreType.DMA,
                pltpu.SemaphoreType.DMA,
            ],
        )(_sc_body))
    return _sc_call_cache[0]


def kernel(x, pos_table):
    Bx, n, d = x.shape
    x2 = jnp.transpose(x, (1, 2, 0)).reshape(R, B)
    posf = pos_table[:n].reshape(R)
    pb16 = jnp.repeat(posf, L)
    out2 = _get_sc_call()(x2, pb16)
    return jnp.transpose(out2.reshape(n, d, Bx), (2, 0, 1))
